# baseline (speedup 1.0000x reference)
import functools

import jax
import jax.numpy as jnp
from jax import lax
from jax.experimental import pallas as pl
from jax.experimental.pallas import tpu as pltpu
from jax.experimental.pallas import tpu_sc as plsc

_NC = 2
_NS = 16
_NW = _NC * _NS
_L = 16
_COLS = 2048 // _NW
_KG = _COLS // _L
_R = 256
_UNROLL = 16


def _sc_body(f_hbm, l_hbm, c_hbm, out_hbm, cbuf, fbuf, lbuf, stage, fsem, lsem):
    wid = lax.axis_index("s") * _NC + lax.axis_index("c")
    col0 = wid * _COLS
    nrows = f_hbm.shape[0]
    nchunk = nrows // _R

    pltpu.sync_copy(c_hbm.at[:, pl.ds(col0, _COLS)], cbuf)

    def _issue(t, b):
        pltpu.make_async_copy(
            f_hbm.at[pl.ds(t * _R, _R), pl.ds(col0, _COLS)],
            fbuf.at[b], fsem.at[b]).start()
        pltpu.make_async_copy(
            l_hbm.at[pl.ds(t * _R, _R)], lbuf.at[b], lsem.at[b]).start()

    def _wait(b):
        pltpu.make_async_copy(
            f_hbm.at[pl.ds(0, _R), pl.ds(col0, _COLS)],
            fbuf.at[b], fsem.at[b]).wait()
        pltpu.make_async_copy(
            l_hbm.at[pl.ds(0, _R)], lbuf.at[b], lsem.at[b]).wait()

    _issue(0, 0)
    _issue(1, 1)

    def _chunk(i, acc):
        for b in range(2):
            t = 2 * i + b
            _wait(b)

            def _rows(j, acc_in):
                accs = list(acc_in)
                lv = lbuf[b, pl.ds(j * _UNROLL, _L)]
                for u in range(_UNROLL):
                    r = j * _UNROLL + u
                    lab = lv[u]
                    for k in range(_KG):
                        cv = cbuf[lab, pl.ds(k * _L, _L)]
                        fv = fbuf[b, r, pl.ds(k * _L, _L)]
                        d = fv - cv
                        accs[k] = accs[k] + d * d
                return tuple(accs)

            acc = lax.fori_loop(0, _R // _UNROLL, _rows, acc)

            @pl.when(t + 2 < nchunk)
            def _():
                _issue(t + 2, b)
        return acc

    zero = jnp.zeros((_L,), jnp.float32)
    acc = lax.fori_loop(0, nchunk // 2, _chunk, (zero,) * _KG)

    stage[...] = acc[0] + acc[1] + acc[2] + acc[3]
    pltpu.sync_copy(stage, out_hbm.at[wid])


def _sc_loss_partials(features, labels, centers):
    mesh = plsc.VectorSubcoreMesh(
        core_axis_name="c", subcore_axis_name="s",
        num_cores=_NC, num_subcores=_NS)
    nclass = centers.shape[0]
    run = pl.kernel(
        _sc_body,
        out_type=jax.ShapeDtypeStruct((_NW, _L), jnp.float32),
        mesh=mesh,
        scratch_types=[
            pltpu.VMEM((nclass, _COLS), jnp.float32),
            pltpu.VMEM((2, _R, _COLS), jnp.float32),
            pltpu.VMEM((2, _R), jnp.int32),
            pltpu.VMEM((_L,), jnp.float32),
            pltpu.SemaphoreType.DMA((2,)),
            pltpu.SemaphoreType.DMA((2,)),
        ],
        compiler_params=pltpu.CompilerParams(use_tc_tiling_on_sc=False),
    )
    return run(features, labels, centers)


def kernel(features, labels, centers):
    batch = features.shape[0]
    partials = _sc_loss_partials(features, labels.astype(jnp.int32), centers)
    return (0.5 / batch) * jnp.sum(partials)


# speedup vs baseline: 1.0992x; 1.0992x over previous
import functools

import jax
import jax.numpy as jnp
from jax import lax
from jax.experimental import pallas as pl
from jax.experimental.pallas import tpu as pltpu
from jax.experimental.pallas import tpu_sc as plsc

_NC = 2
_NS = 16
_NW = _NC * _NS
_L = 16
_COLS = 2048 // _NW
_KG = _COLS // _L
_R = 256
_UNROLL = 16


def _sc_body(f_hbm, l_hbm, c_hbm, out_hbm, cbuf, fbuf, lbuf, stage, fsem, lsem):
    wid = lax.axis_index("s") * _NC + lax.axis_index("c")
    col0 = wid * _COLS
    nrows = f_hbm.shape[0]
    nchunk = nrows // _R

    pltpu.sync_copy(c_hbm.at[:, pl.ds(col0, _COLS)], cbuf)

    def _issue(t, b):
        pltpu.make_async_copy(
            f_hbm.at[pl.ds(t * _R, _R), pl.ds(col0, _COLS)],
            fbuf.at[b], fsem.at[b]).start()
        pltpu.make_async_copy(
            l_hbm.at[pl.ds(t * _R, _R)], lbuf.at[b], lsem.at[b]).start()

    def _wait(b):
        pltpu.make_async_copy(
            f_hbm.at[pl.ds(0, _R), pl.ds(col0, _COLS)],
            fbuf.at[b], fsem.at[b]).wait()
        pltpu.make_async_copy(
            l_hbm.at[pl.ds(0, _R)], lbuf.at[b], lsem.at[b]).wait()

    _issue(0, 0)
    _issue(1, 1)

    def _chunk(i, acc):
        for b in range(2):
            t = 2 * i + b
            _wait(b)

            def _rows(j, acc_in):
                accs = list(acc_in)
                lv = lbuf[b, pl.ds(j * _UNROLL, _L)]
                for u in range(_UNROLL):
                    r = j * _UNROLL + u
                    lab = lv[u]
                    for k in range(_KG):
                        cv = cbuf[lab, pl.ds(k * _L, _L)]
                        fv = fbuf[b, r, pl.ds(k * _L, _L)]
                        d = fv - cv
                        accs[k] = accs[k] + d * d
                return tuple(accs)

            acc = lax.fori_loop(0, _R // _UNROLL, _rows, acc)

            @pl.when(t + 2 < nchunk)
            def _():
                _issue(t + 2, b)
        return acc

    zero = jnp.zeros((_L,), jnp.float32)
    acc = lax.fori_loop(0, nchunk // 2, _chunk, (zero,) * _KG)

    stage[...] = acc[0] + acc[1] + acc[2] + acc[3]
    pltpu.sync_copy(stage, out_hbm.at[wid])


def _sc_loss_partials(features, labels, centers):
    mesh = plsc.VectorSubcoreMesh(
        core_axis_name="c", subcore_axis_name="s",
        num_cores=_NC, num_subcores=_NS)
    nclass = centers.shape[0]
    run = pl.kernel(
        _sc_body,
        out_type=jax.ShapeDtypeStruct((_NW, _L), jnp.float32),
        mesh=mesh,
        scratch_types=[
            pltpu.VMEM((nclass, _COLS), jnp.float32),
            pltpu.VMEM((2, _R, _COLS), jnp.float32),
            pltpu.VMEM((2, _R), jnp.int32),
            pltpu.VMEM((_L,), jnp.float32),
            pltpu.SemaphoreType.DMA((2,)),
            pltpu.SemaphoreType.DMA((2,)),
        ],
        compiler_params=pltpu.CompilerParams(use_tc_tiling_on_sc=False),
    )
    return run(features, labels, centers)


_BB = 1024
_CPAD = 1024


def _tc_block_kernel(lab_ref, f_ref, c_ref, out_ref):
    i = pl.program_id(0)
    lab = lab_ref[0]
    col = jax.lax.broadcasted_iota(jnp.int32, (_BB, _CPAD), 1)
    onehot = (col == lab).astype(jnp.bfloat16)
    bc = jnp.dot(onehot, c_ref[...], preferred_element_type=jnp.float32)
    d = f_ref[...] - bc
    part = jnp.sum(d * d, keepdims=True)

    @pl.when(i == 0)
    def _init():
        out_ref[...] = jnp.zeros((1, 1), jnp.float32)

    out_ref[...] += part


def _tc_loss_sum(features, labels, centers):
    nrows, feat = features.shape
    nclass = centers.shape[0]
    g = nrows // _BB
    lab3 = labels.reshape(g, _BB, 1)
    cpad = jnp.pad(centers, ((0, _CPAD - nclass), (0, 0))).astype(jnp.bfloat16)
    total = pl.pallas_call(
        _tc_block_kernel,
        grid=(g,),
        in_specs=[
            pl.BlockSpec((1, _BB, 1), lambda i: (i, 0, 0)),
            pl.BlockSpec((_BB, feat), lambda i: (i, 0)),
            pl.BlockSpec((_CPAD, feat), lambda i: (0, 0)),
        ],
        out_specs=pl.BlockSpec((1, 1), lambda i: (0, 0)),
        out_shape=jax.ShapeDtypeStruct((1, 1), jnp.float32),
    )(lab3, features, cpad)
    return total[0, 0]


_SC_ROWS = 6144


def kernel(features, labels, centers):
    batch = features.shape[0]
    labels = labels.astype(jnp.int32)
    sc_part = _sc_loss_partials(features[:_SC_ROWS], labels[:_SC_ROWS], centers)
    tc_part = _tc_loss_sum(features[_SC_ROWS:], labels[_SC_ROWS:], centers)
    return (0.5 / batch) * (jnp.sum(sc_part) + tc_part)


# speedup vs baseline: 1.5022x; 1.3666x over previous
import functools

import jax
import jax.numpy as jnp
from jax import lax
from jax.experimental import pallas as pl
from jax.experimental.pallas import tpu as pltpu
from jax.experimental.pallas import tpu_sc as plsc

_NC = 2
_NS = 16
_NW = _NC * _NS
_L = 16
_COLS = 2048 // _NW
_KG = _COLS // _L
_R = 256
_UNROLL = 16


def _sc_body(f_hbm, l_hbm, c_hbm, out_hbm, cbuf, fbuf, lbuf, stage, fsem, lsem):
    wid = lax.axis_index("s") * _NC + lax.axis_index("c")
    col0 = wid * _COLS
    nrows = f_hbm.shape[0]
    nchunk = nrows // _R

    pltpu.sync_copy(c_hbm.at[:, pl.ds(col0, _COLS)], cbuf)

    def _issue(t, b):
        pltpu.make_async_copy(
            f_hbm.at[pl.ds(t * _R, _R), pl.ds(col0, _COLS)],
            fbuf.at[b], fsem.at[b]).start()
        pltpu.make_async_copy(
            l_hbm.at[pl.ds(t * _R, _R)], lbuf.at[b], lsem.at[b]).start()

    def _wait(b):
        pltpu.make_async_copy(
            f_hbm.at[pl.ds(0, _R), pl.ds(col0, _COLS)],
            fbuf.at[b], fsem.at[b]).wait()
        pltpu.make_async_copy(
            l_hbm.at[pl.ds(0, _R)], lbuf.at[b], lsem.at[b]).wait()

    _issue(0, 0)
    _issue(1, 1)

    def _chunk(i, acc):
        for b in range(2):
            t = 2 * i + b
            _wait(b)

            def _rows(j, acc_in):
                accs = list(acc_in)
                lv = lbuf[b, pl.ds(j * _UNROLL, _L)]
                for u in range(_UNROLL):
                    r = j * _UNROLL + u
                    lab = lv[u]
                    for k in range(_KG):
                        cv = cbuf[lab, pl.ds(k * _L, _L)]
                        fv = fbuf[b, r, pl.ds(k * _L, _L)]
                        d = fv - cv
                        accs[k] = accs[k] + d * d
                return tuple(accs)

            acc = lax.fori_loop(0, _R // _UNROLL, _rows, acc)

            @pl.when(t + 2 < nchunk)
            def _():
                _issue(t + 2, b)
        return acc

    zero = jnp.zeros((_L,), jnp.float32)
    acc = lax.fori_loop(0, nchunk // 2, _chunk, (zero,) * _KG)

    stage[...] = acc[0] + acc[1] + acc[2] + acc[3]
    pltpu.sync_copy(stage, out_hbm.at[wid])


def _sc_loss_partials(features, labels, centers):
    mesh = plsc.VectorSubcoreMesh(
        core_axis_name="c", subcore_axis_name="s",
        num_cores=_NC, num_subcores=_NS)
    nclass = centers.shape[0]
    run = pl.kernel(
        _sc_body,
        out_type=jax.ShapeDtypeStruct((_NW, _L), jnp.float32),
        mesh=mesh,
        scratch_types=[
            pltpu.VMEM((nclass, _COLS), jnp.float32),
            pltpu.VMEM((2, _R, _COLS), jnp.float32),
            pltpu.VMEM((2, _R), jnp.int32),
            pltpu.VMEM((_L,), jnp.float32),
            pltpu.SemaphoreType.DMA((2,)),
            pltpu.SemaphoreType.DMA((2,)),
        ],
        compiler_params=pltpu.CompilerParams(use_tc_tiling_on_sc=False),
    )
    return run(features, labels, centers)


_BB = 1024
_CPAD = 1024


def _tc_block_kernel(lab_ref, f_ref, c_ref, out_ref):
    i = pl.program_id(0)
    lab = lab_ref[0]
    col = jax.lax.broadcasted_iota(jnp.int32, (_BB, _CPAD), 1)
    onehot = (col == lab).astype(jnp.bfloat16)
    bc = jnp.dot(onehot, c_ref[...], preferred_element_type=jnp.float32)
    d = f_ref[...] - bc
    part = jnp.sum(d * d, keepdims=True)

    @pl.when(i == 0)
    def _init():
        out_ref[...] = jnp.zeros((1, 1), jnp.float32)

    out_ref[...] += part


def _tc_loss_sum(features, labels, centers, row0):
    nrows, feat = features.shape
    nclass = centers.shape[0]
    g = (nrows - row0) // _BB
    g0 = row0 // _BB
    lab3 = labels.reshape(nrows // _BB, _BB, 1)
    cpad = jnp.pad(centers, ((0, _CPAD - nclass), (0, 0))).astype(jnp.bfloat16)
    total = pl.pallas_call(
        _tc_block_kernel,
        grid=(g,),
        in_specs=[
            pl.BlockSpec((1, _BB, 1), lambda i: (i + g0, 0, 0)),
            pl.BlockSpec((_BB, feat), lambda i: (i + g0, 0)),
            pl.BlockSpec((_CPAD, feat), lambda i: (0, 0)),
        ],
        out_specs=pl.BlockSpec((1, 1), lambda i: (0, 0)),
        out_shape=jax.ShapeDtypeStruct((1, 1), jnp.float32),
    )(lab3, features, cpad)
    return total[0, 0]


_SC_ROWS = 6144


def kernel(features, labels, centers):
    batch = features.shape[0]
    labels = labels.astype(jnp.int32)
    sc_part = _sc_loss_partials(features[:_SC_ROWS], labels[:_SC_ROWS], centers)
    tc_part = _tc_loss_sum(features, labels, centers, _SC_ROWS)
    return (0.5 / batch) * (jnp.sum(sc_part) + tc_part)


# speedup vs baseline: 2.8142x; 1.8734x over previous
import jax
import jax.numpy as jnp
from jax.experimental import pallas as pl

_BB = 1024
_CPAD = 1024


def _block_kernel(lab_ref, f_ref, c_ref, out_ref):
    i = pl.program_id(0)
    lab = lab_ref[0].astype(jnp.int16)
    col = jax.lax.broadcasted_iota(jnp.int16, (_BB, _CPAD), 1)
    onehot = (col == lab).astype(jnp.bfloat16)
    nsp = 8
    ch = f_ref.shape[1] // nsp
    part = jnp.zeros((1, 1), jnp.float32)
    for n in range(nsp):
        bc = jnp.dot(onehot, c_ref[:, n * ch:(n + 1) * ch],
                     preferred_element_type=jnp.float32)
        d = f_ref[:, n * ch:(n + 1) * ch] - bc
        part = part + jnp.sum(d * d, keepdims=True)

    @pl.when(i == 0)
    def _init():
        out_ref[...] = jnp.zeros((1, 1), jnp.float32)

    out_ref[...] += part


def kernel(features, labels, centers):
    batch, feat = features.shape
    nclass = centers.shape[0]
    g = batch // _BB
    lab3 = labels.astype(jnp.int32).reshape(g, _BB, 1)
    cpad = jnp.pad(centers, ((0, _CPAD - nclass), (0, 0))).astype(jnp.bfloat16)
    total = pl.pallas_call(
        _block_kernel,
        grid=(g,),
        in_specs=[
            pl.BlockSpec((1, _BB, 1), lambda i: (i, 0, 0)),
            pl.BlockSpec((_BB, feat), lambda i: (i, 0)),
            pl.BlockSpec((_CPAD, feat), lambda i: (0, 0)),
        ],
        out_specs=pl.BlockSpec((1, 1), lambda i: (0, 0)),
        out_shape=jax.ShapeDtypeStruct((1, 1), jnp.float32),
    )(lab3, features, cpad)
    return (0.5 / batch) * total[0, 0]
